# baseline (speedup 1.0000x reference)
import functools

import jax
import jax.numpy as jnp
from jax import lax
from jax.experimental import pallas as pl
from jax.experimental.pallas import tpu as pltpu
from jax.experimental.pallas import tpu_sc as plsc

_B, _S, _C, _M = 16, 512, 256, 2048
_NW = 32
_FPW = _M * _B // _NW
_FCH = 256
_NCH = _FPW // _FCH
_W = 128
_L = 16


def _sc_body(xpad_hbm, dur_hbm, ml_hbm, out_hbm, mask_hbm,
             dur_v, cs_v, ml_v, idx_v, msk_v, rowbuf, outa, outb,
             gsem, w0, w1):
    wid = lax.axis_index("s") * 2 + lax.axis_index("c")
    b = wid // 2
    fb = (wid % 2) * _FPW
    b513 = b * (_S + 1)

    pltpu.sync_copy(ml_hbm, ml_v)
    pltpu.sync_copy(dur_hbm.at[b], dur_v)
    mlv = ml_v[...]

    carry = jnp.int32(0)
    for j in range(_S // _L):
        d_i = dur_v[pl.ds(j * _L, _L)].astype(jnp.int32)
        cc = jnp.minimum(plsc.cumsum(d_i) + carry, mlv)
        cs_v[pl.ds(j * _L, _L)] = cc
        carry = carry + jnp.sum(d_i)
    total_v = jnp.minimum(jnp.full((_L,), carry, jnp.int32), mlv)

    lane = lax.iota(jnp.int32, _L)
    zrow = jnp.int32(b513 + _S)

    def _index_all(v, _):
        t = fb + v * _L + lane
        lo = jnp.zeros((_L,), jnp.int32)
        hi = jnp.full((_L,), _S, jnp.int32)
        for _u in range(10):
            mid = (lo + hi) >> 1
            le = plsc.load_gather(cs_v, [mid]) <= t
            lo = jnp.where(le, mid + 1, lo)
            hi = jnp.where(le, hi, mid)
        valid = t < total_v
        g = jnp.where(valid, b513 + lo, zrow)
        idx_v[pl.ds(v * _L, _L)] = g
        msk_v[pl.ds(v * _L, _L)] = (~valid).astype(jnp.int32)
        return 0

    lax.fori_loop(0, _FPW // _L, _index_all, 0)

    outs, wsems = [outa, outb], [w0, w1]
    wh = [None] * (_FPW // _W)
    obase = wid * _FPW
    for c in range(_NCH):
        first = idx_v[pl.ds(c * _FCH, _L)]
        last = idx_v[pl.ds(c * _FCH + _FCH - _L, _L)]
        lo_g = jnp.min(first)
        hi_g = jnp.max(last)
        abase = pl.multiple_of(
            jnp.minimum(lo_g, b513 + _S + 1 - _W) & ~jnp.int32(7), 8)
        linear_ok = (hi_g - abase) <= _W - 1

        @pl.when(linear_ok)
        def _read_window():
            pltpu.sync_copy(xpad_hbm.at[pl.ds(abase, _W)], rowbuf)

        for h in range(_FCH // _W):
            half = c * (_FCH // _W) + h
            obuf = outs[half % 2]
            if half >= 2:
                wh[half - 2].wait()

            @pl.when(linear_ok)
            def _expand():
                for v in range(_W // _L):
                    jloc = idx_v[pl.ds(c * _FCH + h * _W + v * _L, _L)] - abase
                    rvec = v * _L + lane

                    @plsc.parallel_loop(0, _C, unroll=8)
                    def _wbody(w):
                        wv = jnp.full((_L,), w, jnp.int32)
                        val = plsc.load_gather(rowbuf, [jloc, wv])
                        plsc.store_scatter(obuf, [rvec, wv], val)

            @pl.when(jnp.logical_not(linear_ok))
            def _gather_fallback():
                pltpu.async_copy(
                    xpad_hbm.at[idx_v.at[pl.ds((c * 2 + h) * _W, _W)]],
                    obuf, gsem).wait()

            wh[half] = pltpu.async_copy(
                obuf, out_hbm.at[pl.ds(obase + half * _W, _W)],
                wsems[half % 2])
    wh[-2].wait()
    wh[-1].wait()

    pltpu.sync_copy(msk_v, mask_hbm.at[pl.ds(obase, _FPW)])


_sc_expand = functools.partial(
    pl.kernel,
    mesh=plsc.VectorSubcoreMesh(core_axis_name="c", subcore_axis_name="s"),
    out_type=[
        jax.ShapeDtypeStruct((_B * _M, _C), jnp.float32),
        jax.ShapeDtypeStruct((_B * _M,), jnp.int32),
    ],
    scratch_types=[
        pltpu.VMEM((_S,), jnp.float32),
        pltpu.VMEM((_S,), jnp.int32),
        pltpu.VMEM((_L,), jnp.int32),
        pltpu.VMEM((_FPW,), jnp.int32),
        pltpu.VMEM((_FPW,), jnp.int32),
        pltpu.VMEM((_W, _C), jnp.float32),
        pltpu.VMEM((_W, _C), jnp.float32),
        pltpu.VMEM((_W, _C), jnp.float32),
        pltpu.SemaphoreType.DMA,
        pltpu.SemaphoreType.DMA,
        pltpu.SemaphoreType.DMA,
    ],
    compiler_params=pltpu.CompilerParams(needs_layout_passes=False),
)(_sc_body)


def kernel(x, durations, max_length):
    B, S, C = x.shape
    xpad = jnp.concatenate(
        [x, jnp.zeros((B, 1, C), x.dtype)], axis=1).reshape(B * (S + 1), C)
    d = jnp.round(durations)
    ml = jnp.full((_L,), max_length, jnp.int32)
    out, mask_i = _sc_expand(xpad, d, ml)
    expanded = out.reshape(B, _M, C)
    mel_masks = mask_i.reshape(B, _M).astype(bool)
    return expanded, mel_masks


# speedup vs baseline: 1.9969x; 1.9969x over previous
import functools

import jax
import jax.numpy as jnp
from jax import lax
from jax.experimental import pallas as pl
from jax.experimental.pallas import tpu as pltpu
from jax.experimental.pallas import tpu_sc as plsc

_B, _S, _C, _M = 16, 512, 256, 2048
_NW = 32
_FPW = _M * _B // _NW
_CH = 128
_NCH = _FPW // _CH
_L = 16


def _sc_body(xpad_hbm, dur_hbm, ml_hbm, out_hbm, mask_hbm,
             dur_v, cs_v, ml_v, idx_v, msk_v, bufa, bufb,
             ga, gb, wa, wb):
    wid = lax.axis_index("s") * 2 + lax.axis_index("c")
    b = wid // 2
    fb = (wid % 2) * _FPW

    pltpu.sync_copy(ml_hbm, ml_v)
    pltpu.sync_copy(dur_hbm.at[b], dur_v)
    mlv = ml_v[...]

    carry = jnp.int32(0)
    for j in range(_S // _L):
        d_i = dur_v[pl.ds(j * _L, _L)].astype(jnp.int32)
        cc = jnp.minimum(plsc.cumsum(d_i) + carry, mlv)
        cs_v[pl.ds(j * _L, _L)] = cc
        carry = carry + jnp.sum(d_i)
    total_v = jnp.minimum(jnp.full((_L,), carry, jnp.int32), mlv)

    lane = lax.iota(jnp.int32, _L)
    zrow = jnp.int32(b * (_S + 1) + _S)

    def _index_chunk(c):
        def body(v, _):
            t = fb + c * _CH + v * _L + lane
            lo = jnp.zeros((_L,), jnp.int32)
            hi = jnp.full((_L,), _S, jnp.int32)
            for _ in range(10):
                mid = (lo + hi) >> 1
                le = plsc.load_gather(cs_v, [mid]) <= t
                lo = jnp.where(le, mid + 1, lo)
                hi = jnp.where(le, hi, mid)
            valid = t < total_v
            g = jnp.where(valid, b * (_S + 1) + lo, zrow)
            idx_v[pl.ds(c * _CH + v * _L, _L)] = g
            msk_v[pl.ds(c * _CH + v * _L, _L)] = (~valid).astype(jnp.int32)
            return 0

        lax.fori_loop(0, _CH // _L, body, 0)

    bufs, gsems, wsems = [bufa, bufb], [ga, gb], [wa, wb]
    gh = [None] * _NCH
    wh = [None] * _NCH
    obase = wid * _FPW
    for c in range(_NCH):
        _index_chunk(c)
        if c >= 2:
            wh[c - 2].wait()
        gh[c] = pltpu.async_copy(
            xpad_hbm.at[idx_v.at[pl.ds(c * _CH, _CH)]], bufs[c % 2],
            gsems[c % 2])
        if c >= 1:
            gh[c - 1].wait()
            wh[c - 1] = pltpu.async_copy(
                bufs[(c - 1) % 2], out_hbm.at[pl.ds(obase + (c - 1) * _CH, _CH)],
                wsems[(c - 1) % 2])
    gh[_NCH - 1].wait()
    wh[_NCH - 1] = pltpu.async_copy(
        bufs[(_NCH - 1) % 2],
        out_hbm.at[pl.ds(obase + (_NCH - 1) * _CH, _CH)],
        wsems[(_NCH - 1) % 2])
    wh[_NCH - 2].wait()
    wh[_NCH - 1].wait()

    pltpu.sync_copy(msk_v, mask_hbm.at[pl.ds(obase, _FPW)])


_sc_expand = functools.partial(
    pl.kernel,
    mesh=plsc.VectorSubcoreMesh(core_axis_name="c", subcore_axis_name="s"),
    out_type=[
        jax.ShapeDtypeStruct((_B * _M, _C), jnp.float32),
        jax.ShapeDtypeStruct((_B * _M,), jnp.int32),
    ],
    scratch_types=[
        pltpu.VMEM((_S,), jnp.float32),
        pltpu.VMEM((_S,), jnp.int32),
        pltpu.VMEM((_L,), jnp.int32),
        pltpu.VMEM((_FPW,), jnp.int32),
        pltpu.VMEM((_FPW,), jnp.int32),
        pltpu.VMEM((_CH, _C), jnp.float32),
        pltpu.VMEM((_CH, _C), jnp.float32),
        pltpu.SemaphoreType.DMA,
        pltpu.SemaphoreType.DMA,
        pltpu.SemaphoreType.DMA,
        pltpu.SemaphoreType.DMA,
    ],
    compiler_params=pltpu.CompilerParams(needs_layout_passes=False, use_tc_tiling_on_sc=False),
)(_sc_body)


def kernel(x, durations, max_length):
    B, S, C = x.shape
    xpad = jnp.concatenate(
        [x, jnp.zeros((B, 1, C), x.dtype)], axis=1).reshape(B * (S + 1), C)
    d = jnp.round(durations)
    ml = jnp.full((_L,), max_length, jnp.int32)
    out, mask_i = _sc_expand(xpad, d, ml)
    expanded = out.reshape(B, _M, C)
    mel_masks = mask_i.reshape(B, _M).astype(bool)
    return expanded, mel_masks


# speedup vs baseline: 2.7472x; 1.3758x over previous
import functools

import jax
import jax.numpy as jnp
from jax import lax
from jax.experimental import pallas as pl
from jax.experimental.pallas import tpu as pltpu
from jax.experimental.pallas import tpu_sc as plsc

_B, _S, _C, _M = 16, 512, 256, 2048
_NW = 32
_FPW = _M * _B // _NW
_CH = 128
_NCH = _FPW // _CH
_L = 16


def _sc_body(xpad_hbm, dur_hbm, ml_hbm, out_hbm, mask_hbm,
             dur_v, cs_v, ml_v, idx_v, msk_v, bufa, bufb,
             ga, gb, wa, wb):
    wid = lax.axis_index("s") * 2 + lax.axis_index("c")
    b = wid // 2
    fb = (wid % 2) * _FPW

    pltpu.sync_copy(ml_hbm, ml_v)
    pltpu.sync_copy(dur_hbm.at[b], dur_v)
    mlv = ml_v[...]

    carry = jnp.int32(0)
    for j in range(_S // _L):
        d_i = dur_v[pl.ds(j * _L, _L)].astype(jnp.int32)
        cc = jnp.minimum(plsc.cumsum(d_i) + carry, mlv)
        cs_v[pl.ds(j * _L, _L)] = cc
        carry = carry + jnp.sum(d_i)
    total_v = jnp.minimum(jnp.full((_L,), carry, jnp.int32), mlv)

    lane = lax.iota(jnp.int32, _L)
    zrow = jnp.int32(b * (_S + 1) + _S)

    def _index_chunk(c):
        def body(v, _):
            t = fb + c * _CH + v * _L + lane
            lo = jnp.zeros((_L,), jnp.int32)
            hi = jnp.full((_L,), _S, jnp.int32)
            for _ in range(10):
                mid = (lo + hi) >> 1
                le = plsc.load_gather(cs_v, [mid]) <= t
                lo = jnp.where(le, mid + 1, lo)
                hi = jnp.where(le, hi, mid)
            valid = t < total_v
            g = jnp.where(valid, b * (_S + 1) + lo, zrow)
            idx_v[pl.ds(c * _CH + v * _L, _L)] = g
            msk_v[pl.ds(c * _CH + v * _L, _L)] = (~valid).astype(jnp.int32)
            return 0

        lax.fori_loop(0, _CH // _L, body, 0)

    bufs, gsems, wsems = [bufa, bufb], [ga, gb], [wa, wb]
    gh = [None] * _NCH
    wh = [None] * _NCH
    obase = wid * _FPW
    for c in range(_NCH):
        _index_chunk(c)
        if c >= 2:
            wh[c - 2].wait()
        gh[c] = pltpu.async_copy(
            xpad_hbm.at[idx_v.at[pl.ds(c * _CH, _CH)]], bufs[c % 2],
            gsems[c % 2])
        if c >= 1:
            gh[c - 1].wait()
            wh[c - 1] = pltpu.async_copy(
                bufs[(c - 1) % 2], out_hbm.at[pl.ds(obase + (c - 1) * _CH, _CH)],
                wsems[(c - 1) % 2])
    gh[_NCH - 1].wait()
    wh[_NCH - 1] = pltpu.async_copy(
        bufs[(_NCH - 1) % 2],
        out_hbm.at[pl.ds(obase + (_NCH - 1) * _CH, _CH)],
        wsems[(_NCH - 1) % 2])
    wh[_NCH - 2].wait()
    wh[_NCH - 1].wait()

    pltpu.sync_copy(msk_v, mask_hbm.at[pl.ds(obase, _FPW)])


_sc_expand = functools.partial(
    pl.kernel,
    mesh=plsc.VectorSubcoreMesh(core_axis_name="c", subcore_axis_name="s"),
    out_type=[
        jax.ShapeDtypeStruct((_B * _M, _C), jnp.float32),
        jax.ShapeDtypeStruct((_B * _M,), jnp.int32),
    ],
    scratch_types=[
        pltpu.VMEM((_S,), jnp.float32),
        pltpu.VMEM((_S,), jnp.int32),
        pltpu.VMEM((_L,), jnp.int32),
        pltpu.VMEM((_FPW,), jnp.int32),
        pltpu.VMEM((_FPW,), jnp.int32),
        pltpu.VMEM((_CH, _C), jnp.float32),
        pltpu.VMEM((_CH, _C), jnp.float32),
        pltpu.SemaphoreType.DMA,
        pltpu.SemaphoreType.DMA,
        pltpu.SemaphoreType.DMA,
        pltpu.SemaphoreType.DMA,
    ],
    compiler_params=pltpu.CompilerParams(needs_layout_passes=False),
)(_sc_body)


def kernel(x, durations, max_length):
    B, S, C = x.shape
    xpad = jnp.concatenate(
        [x, jnp.zeros((B, 1, C), x.dtype)], axis=1).reshape(B * (S + 1), C)
    d = jnp.round(durations)
    ml = jnp.full((_L,), max_length, jnp.int32)
    out, mask_i = _sc_expand(xpad, d, ml)
    expanded = out.reshape(B, _M, C)
    mel_masks = mask_i.reshape(B, _M).astype(bool)
    return expanded, mel_masks
